# baseline (speedup 1.0000x reference)
import jax
import jax.numpy as jnp
from jax import lax
from jax.experimental import pallas as pl
from jax.experimental.pallas import tpu as pltpu
from jax.experimental.pallas import tpu_sc as plsc

B = 16384
D = 64
L = 16
NC = 2
NS = 16
NW = NC * NS
BPW = B // NW
CHUNK = 256
NCH = BPW // CHUNK
NGC = CHUNK // L


def _mf_body(uid_hbm, mid_hbm, uemb_hbm, memb_hbm, ubias_hbm, mbias_hbm,
             out_hbm,
             uid_v, mid_v, uhalf_v, mhalf_v,
             uidx2, midx2,
             urows_v, mrows_v, ub_v, mb_v, out_v, sem):
    wid = lax.axis_index("s") * NC + lax.axis_index("c")
    base = wid * BPW

    pltpu.sync_copy(uid_hbm.at[pl.ds(base, BPW)], uid_v)
    pltpu.sync_copy(mid_hbm.at[pl.ds(base, BPW)], mid_v)

    bias_cps = [
        pltpu.async_copy(ubias_hbm.at[uid_v], ub_v, sem),
        pltpu.async_copy(mbias_hbm.at[mid_v], mb_v, sem),
    ]

    for ci in range(NCH):
        def idx_body(g, carry, ci=ci):
            rc0 = pl.multiple_of(g * L, L)
            r0 = pl.multiple_of(ci * CHUNK + rc0, L)
            sl = pl.ds(r0, L)
            u = uid_v[sl]
            m = mid_v[sl]
            cs = pl.ds(rc0, L)
            uidx2[ci][cs] = lax.shift_right_logical(u, 1)
            midx2[ci][cs] = lax.shift_right_logical(m, 1)
            uhalf_v[sl] = lax.shift_left(jnp.bitwise_and(u, 1), 6)
            mhalf_v[sl] = lax.shift_left(jnp.bitwise_and(m, 1), 6)
            return carry

        lax.fori_loop(0, NGC, idx_body, 0)

    lanes = lax.iota(jnp.int32, L)

    for c in range(NCH):
        cps = [
            pltpu.async_copy(uemb_hbm.at[uidx2[c]], urows_v, sem),
            pltpu.async_copy(memb_hbm.at[midx2[c]], mrows_v, sem),
        ]
        for cp in cps:
            cp.wait()

        def group_body(g, carry, c=c):
            rr0 = pl.multiple_of(g * L, L)
            r0 = pl.multiple_of(c * CHUNK + rr0, L)
            hu_vec = uhalf_v[pl.ds(r0, L)]
            hm_vec = mhalf_v[pl.ds(r0, L)]
            dots = jnp.zeros((L,), jnp.float32)
            for j in range(L):
                rr = rr0 + j
                hu = pl.multiple_of(hu_vec[j], L)
                hm = pl.multiple_of(hm_vec[j], L)
                p0 = urows_v[rr, pl.ds(hu, L)] * mrows_v[rr, pl.ds(hm, L)]
                p1 = urows_v[rr, pl.ds(hu + L, L)] * mrows_v[rr, pl.ds(hm + L, L)]
                p2 = urows_v[rr, pl.ds(hu + 2 * L, L)] * mrows_v[rr, pl.ds(hm + 2 * L, L)]
                p3 = urows_v[rr, pl.ds(hu + 3 * L, L)] * mrows_v[rr, pl.ds(hm + 3 * L, L)]
                s = jnp.sum((p0 + p1) + (p2 + p3))
                dots = jnp.where(lanes == j, s, dots)
            out_v[pl.ds(r0, L)] = dots
            return carry

        lax.fori_loop(0, NGC, group_body, 0)

    for cp in bias_cps:
        cp.wait()

    def bias_body(g, carry):
        sl = pl.ds(pl.multiple_of(g * L, L), L)
        out_v[sl] = out_v[sl] + ub_v[sl] + mb_v[sl]
        return carry

    lax.fori_loop(0, BPW // L, bias_body, 0)

    pltpu.sync_copy(out_v, out_hbm.at[pl.ds(base, BPW)])


@jax.jit
def _mf_call(user_ids, movie_ids, user_emb, movie_emb, user_bias, movie_bias):
    mesh = plsc.VectorSubcoreMesh(core_axis_name="c", subcore_axis_name="s")
    run = pl.kernel(
        _mf_body,
        mesh=mesh,
        compiler_params=pltpu.CompilerParams(needs_layout_passes=False),
        out_type=jax.ShapeDtypeStruct((B,), jnp.float32),
        scratch_types=[
            pltpu.VMEM((BPW,), jnp.int32),
            pltpu.VMEM((BPW,), jnp.int32),
            pltpu.VMEM((BPW,), jnp.int32),
            pltpu.VMEM((BPW,), jnp.int32),
            [pltpu.VMEM((CHUNK,), jnp.int32) for _ in range(NCH)],
            [pltpu.VMEM((CHUNK,), jnp.int32) for _ in range(NCH)],
            pltpu.VMEM((CHUNK, 2 * D), jnp.float32),
            pltpu.VMEM((CHUNK, 2 * D), jnp.float32),
            pltpu.VMEM((BPW,), jnp.float32),
            pltpu.VMEM((BPW,), jnp.float32),
            pltpu.VMEM((BPW,), jnp.float32),
            pltpu.SemaphoreType.DMA,
        ],
    )
    return run(user_ids, movie_ids, user_emb, movie_emb, user_bias, movie_bias)


def kernel(user_ids, movie_ids, user_emb, movie_emb, user_bias, movie_bias):
    return _mf_call(
        user_ids.astype(jnp.int32),
        movie_ids.astype(jnp.int32),
        user_emb.reshape(-1, 2 * D),
        movie_emb.reshape(-1, 2 * D),
        user_bias.reshape(-1),
        movie_bias.reshape(-1),
    )


# speedup vs baseline: 1.0049x; 1.0049x over previous
import jax
import jax.numpy as jnp
from jax import lax
from jax.experimental import pallas as pl
from jax.experimental.pallas import tpu as pltpu
from jax.experimental.pallas import tpu_sc as plsc

B = 16384
D = 64
L = 16
NC = 2
NS = 16
NW = NC * NS
BPW = B // NW
NG = BPW // L


def _mf_body(uid_hbm, mid_hbm, uemb_hbm, memb_hbm, ubias_hbm, mbias_hbm,
             out_hbm,
             uid_v, mid_v, urows, mrows, ub_v, mb_v, out_v, bias_sem, sem):
    wid = lax.axis_index("s") * NC + lax.axis_index("c")
    base = wid * BPW

    pltpu.sync_copy(uid_hbm.at[pl.ds(base, BPW)], uid_v)
    pltpu.sync_copy(mid_hbm.at[pl.ds(base, BPW)], mid_v)

    cps = [
        pltpu.async_copy(uemb_hbm.at[uid_v], urows, sem),
        pltpu.async_copy(memb_hbm.at[mid_v], mrows, sem),
    ]
    bias_cps = [
        pltpu.async_copy(ubias_hbm.at[uid_v], ub_v, bias_sem),
        pltpu.async_copy(mbias_hbm.at[mid_v], mb_v, bias_sem),
    ]
    for cp in cps:
        cp.wait()

    lanes = lax.iota(jnp.int32, L)

    def group_body(g, carry):
        sl = pl.ds(pl.multiple_of(g * L, L), L)
        acc = jnp.zeros((L,), jnp.float32)
        for j in range(L):
            r = g * L + j
            dotv = jnp.zeros((L,), jnp.float32)
            for q in range(D // L):
                dsq = pl.ds(q * L, L)
                dotv = dotv + urows[r, dsq] * mrows[r, dsq]
            acc = jnp.where(lanes == j, jnp.sum(dotv), acc)
        out_v[sl] = acc
        return carry

    lax.fori_loop(0, NG, group_body, 0)

    for cp in bias_cps:
        cp.wait()

    def bias_body(g, carry):
        sl = pl.ds(pl.multiple_of(g * L, L), L)
        out_v[sl] = out_v[sl] + ub_v[sl] + mb_v[sl]
        return carry

    lax.fori_loop(0, NG, bias_body, 0)

    pltpu.sync_copy(out_v, out_hbm.at[pl.ds(base, BPW)])


@jax.jit
def _mf_call(user_ids, movie_ids, user_emb, movie_emb, user_bias, movie_bias):
    mesh = plsc.VectorSubcoreMesh(core_axis_name="c", subcore_axis_name="s")
    run = pl.kernel(
        _mf_body,
        mesh=mesh,
        compiler_params=pltpu.CompilerParams(
            needs_layout_passes=False,
            use_tc_tiling_on_sc=False,
        ),
        out_type=jax.ShapeDtypeStruct((B,), jnp.float32),
        scratch_types=[
            pltpu.VMEM((BPW,), jnp.int32),
            pltpu.VMEM((BPW,), jnp.int32),
            pltpu.VMEM((BPW, D), jnp.float32),
            pltpu.VMEM((BPW, D), jnp.float32),
            pltpu.VMEM((BPW,), jnp.float32),
            pltpu.VMEM((BPW,), jnp.float32),
            pltpu.VMEM((BPW,), jnp.float32),
            pltpu.SemaphoreType.DMA,
            pltpu.SemaphoreType.DMA,
        ],
    )
    return run(user_ids, movie_ids, user_emb, movie_emb, user_bias, movie_bias)


def kernel(user_ids, movie_ids, user_emb, movie_emb, user_bias, movie_bias):
    return _mf_call(
        user_ids.astype(jnp.int32),
        movie_ids.astype(jnp.int32),
        user_emb,
        movie_emb,
        user_bias.T.reshape(-1),
        movie_bias.T.reshape(-1),
    )
